# Initial kernel scaffold; baseline (speedup 1.0000x reference)
import jax
import jax.numpy as jnp
from jax.experimental import pallas as pl


def kernel(u_emb, i_emb, a_emb, o_emb, s, mlp_ao_W1, mlp_ao_b1, mlp_ao_W2, mlp_ao_b2, mlp_ui_W1, mlp_ui_b1, mlp_ui_W2, mlp_ui_b2):
    raise NotImplementedError("write your pallas kernel here")


# speedup vs baseline: 2.7612x; 2.7612x over previous
import functools

import jax
import jax.numpy as jnp
from jax import lax
from jax.experimental import pallas as pl
from jax.experimental.pallas import tpu as pltpu
from jax.experimental.pallas import tpu_sc as plsc

B, N, D, H1, H2, R = 1024, 8, 512, 512, 256, 8
T = B * N
BLK = 128
NBLK = T // BLK
MT = 256
NMT = T // MT

_NC, _NS = 2, 16
_NW = _NC * _NS
_PW = T // _NW
_CH = 128


def _leaky(x):
    return jnp.where(x >= 0, x, 0.01 * x)


def _prep_body(s_ref, u_ref, i_ref, w1u_ref, w1i_ref, b1_ref, w2_ref, b2_ref,
               dest_ref, offs_ref, ui_ref):
    sv = s_ref[...]
    jj = lax.broadcasted_iota(jnp.int32, (BLK, BLK), 0)
    ii = lax.broadcasted_iota(jnp.int32, (BLK, BLK), 1)
    tri = (jj < ii).astype(jnp.float32)
    tj = lax.broadcasted_iota(jnp.int32, (NBLK, NBLK), 1)
    ti = lax.broadcasted_iota(jnp.int32, (NBLK, NBLK), 0)
    tri_blk = (tj < ti).astype(jnp.float32)
    lane = lax.broadcasted_iota(jnp.int32, (1, 128), 1)

    dest_acc = jnp.zeros((NBLK, BLK), jnp.float32)
    offs_acc = jnp.zeros((1, 128), jnp.float32)
    off = jnp.float32(0.0)
    for r in range(R):
        oh = (sv == r).astype(jnp.float32)
        excl = jnp.dot(oh, tri, preferred_element_type=jnp.float32)
        counts = jnp.sum(oh, axis=1, keepdims=True)
        base = jnp.dot(tri_blk, counts, preferred_element_type=jnp.float32)
        offs_acc = offs_acc + jnp.where(lane == r, off, 0.0)
        dest_acc = dest_acc + oh * (off + base + excl)
        off = off + jnp.sum(counts)
    offs_acc = offs_acc + jnp.where(lane == R, off, 0.0)
    dest_ref[...] = dest_acc.astype(jnp.int32)
    offs_ref[...] = offs_acc.astype(jnp.int32)

    h = jnp.dot(u_ref[...], w1u_ref[...], preferred_element_type=jnp.float32)
    h = h + jnp.dot(i_ref[...], w1i_ref[...], preferred_element_type=jnp.float32)
    h = _leaky(h + b1_ref[...])
    ui = _leaky(jnp.dot(h, w2_ref[...], preferred_element_type=jnp.float32)
                + b2_ref[...])
    ui_ref[...] = ui


def _group_body(offs_ref, xa_ref, xo_ref, w1a_ref, w1o_ref, b1_ref, w2_ref,
                b2_ref, out_ref):
    t = pl.program_id(0)
    row0 = t * MT
    rows = lax.broadcasted_iota(jnp.int32, (MT, 1), 0) + row0
    out_ref[...] = jnp.zeros((MT, H2), jnp.float32)
    for r in range(R):
        lo = offs_ref[r]
        hi = offs_ref[r + 1]

        @pl.when((lo < row0 + MT) & (hi > row0))
        def _():
            h = jnp.dot(xa_ref[...], w1a_ref[r], preferred_element_type=jnp.float32)
            h = h + jnp.dot(xo_ref[...], w1o_ref[r], preferred_element_type=jnp.float32)
            h = _leaky(h + b1_ref[r][None, :])
            g = _leaky(jnp.dot(h, w2_ref[r], preferred_element_type=jnp.float32)
                       + b2_ref[r][None, :])
            mask = (rows >= lo) & (rows < hi)
            out_ref[...] = jnp.where(mask, g, out_ref[...])


def _dot_body(ui_ref, aos_ref, out_ref):
    aos = aos_ref[...].reshape(B, N, H2)
    ui = ui_ref[...]
    out_ref[...] = jnp.sum(aos * ui[:, None, :], axis=-1)


_sc_mesh = plsc.VectorSubcoreMesh(core_axis_name="c", subcore_axis_name="s")


@functools.partial(
    pl.kernel,
    out_type=(
        jax.ShapeDtypeStruct((T, D), jnp.float32),
        jax.ShapeDtypeStruct((T, D), jnp.float32),
    ),
    mesh=_sc_mesh,
    scratch_types=[
        pltpu.VMEM((_CH,), jnp.int32),
        pltpu.VMEM((_CH, D), jnp.float32),
        pltpu.SemaphoreType.DMA,
    ],
)
def _sc_scatter(a_hbm, o_hbm, dest_hbm, xa_hbm, xo_hbm, idx_v, rows_v, sem):
    wid = lax.axis_index("s") * _NC + lax.axis_index("c")
    base = wid * _PW
    for c in range(_PW // _CH):
        cb = base + c * _CH
        pltpu.sync_copy(dest_hbm.at[pl.ds(cb, _CH)], idx_v)
        pltpu.sync_copy(a_hbm.at[pl.ds(cb, _CH)], rows_v)
        pltpu.async_copy(rows_v, xa_hbm.at[idx_v], sem).wait()
        pltpu.sync_copy(o_hbm.at[pl.ds(cb, _CH)], rows_v)
        pltpu.async_copy(rows_v, xo_hbm.at[idx_v], sem).wait()


@functools.partial(
    pl.kernel,
    out_type=jax.ShapeDtypeStruct((T, H2), jnp.float32),
    mesh=_sc_mesh,
    scratch_types=[
        pltpu.VMEM((_CH,), jnp.int32),
        pltpu.VMEM((_CH, H2), jnp.float32),
        pltpu.SemaphoreType.DMA,
    ],
)
def _sc_gather(h2s_hbm, dest_hbm, aos_hbm, idx_v, rows_v, sem):
    wid = lax.axis_index("s") * _NC + lax.axis_index("c")
    base = wid * _PW
    for c in range(_PW // _CH):
        cb = base + c * _CH
        pltpu.sync_copy(dest_hbm.at[pl.ds(cb, _CH)], idx_v)
        pltpu.async_copy(h2s_hbm.at[idx_v], rows_v, sem).wait()
        pltpu.sync_copy(rows_v, aos_hbm.at[pl.ds(cb, _CH)])


def kernel(u_emb, i_emb, a_emb, o_emb, s,
           mlp_ao_W1, mlp_ao_b1, mlp_ao_W2, mlp_ao_b2,
           mlp_ui_W1, mlp_ui_b1, mlp_ui_W2, mlp_ui_b2):
    s2d = s.reshape(NBLK, BLK)
    w1u = mlp_ui_W1[:D]
    w1i = mlp_ui_W1[D:]
    b1 = mlp_ui_b1.reshape(1, H1)
    b2 = mlp_ui_b2.reshape(1, H2)

    dest2d, offs, ui_emb = pl.pallas_call(
        _prep_body,
        out_shape=(
            jax.ShapeDtypeStruct((NBLK, BLK), jnp.int32),
            jax.ShapeDtypeStruct((1, 128), jnp.int32),
            jax.ShapeDtypeStruct((B, H2), jnp.float32),
        ),
    )(s2d, u_emb, i_emb, w1u, w1i, b1, mlp_ui_W2, b2)

    dest = dest2d.reshape(T)
    a2 = a_emb.reshape(T, D)
    o2 = o_emb.reshape(T, D)

    xa, xo = _sc_scatter(a2, o2, dest)

    offs9 = offs[0, :R + 1]
    w1a = mlp_ao_W1[:, :D, :]
    w1o = mlp_ao_W1[:, D:, :]

    h2s = pl.pallas_call(
        _group_body,
        grid=(NMT,),
        in_specs=[
            pl.BlockSpec(memory_space=pltpu.SMEM),
            pl.BlockSpec((MT, D), lambda t: (t, 0)),
            pl.BlockSpec((MT, D), lambda t: (t, 0)),
            pl.BlockSpec((R, D, H1), lambda t: (0, 0, 0)),
            pl.BlockSpec((R, D, H1), lambda t: (0, 0, 0)),
            pl.BlockSpec((R, H1), lambda t: (0, 0)),
            pl.BlockSpec((R, H1, H2), lambda t: (0, 0, 0)),
            pl.BlockSpec((R, H2), lambda t: (0, 0)),
        ],
        out_specs=pl.BlockSpec((MT, H2), lambda t: (t, 0)),
        out_shape=jax.ShapeDtypeStruct((T, H2), jnp.float32),
    )(offs9, xa, xo, w1a, w1o, mlp_ao_b1, mlp_ao_W2, mlp_ao_b2)

    aos = _sc_gather(h2s, dest)

    pred = pl.pallas_call(
        _dot_body,
        out_shape=jax.ShapeDtypeStruct((B, N), jnp.float32),
    )(ui_emb, aos)
    return pred


# speedup vs baseline: 2.8064x; 1.0164x over previous
import functools

import jax
import jax.numpy as jnp
from jax import lax
from jax.experimental import pallas as pl
from jax.experimental.pallas import tpu as pltpu
from jax.experimental.pallas import tpu_sc as plsc

B, N, D, H1, H2, R = 1024, 8, 512, 512, 256, 8
T = B * N
BLK = 128
NBLK = T // BLK
MT = 256
NMT = T // MT

_NC, _NS = 2, 16
_NW = _NC * _NS
_PW = T // _NW
_CH = 128


def _leaky(x):
    return jnp.where(x >= 0, x, 0.01 * x)


def _prep_body(s_ref, u_ref, i_ref, w1u_ref, w1i_ref, b1_ref, w2_ref, b2_ref,
               dest_ref, offs_ref, ui_ref):
    sv = s_ref[...]
    jj = lax.broadcasted_iota(jnp.int32, (BLK, BLK), 0)
    ii = lax.broadcasted_iota(jnp.int32, (BLK, BLK), 1)
    tri = (jj < ii).astype(jnp.float32)
    tj = lax.broadcasted_iota(jnp.int32, (NBLK, NBLK), 1)
    ti = lax.broadcasted_iota(jnp.int32, (NBLK, NBLK), 0)
    tri_blk = (tj < ti).astype(jnp.float32)
    lane = lax.broadcasted_iota(jnp.int32, (1, 128), 1)

    dest_acc = jnp.zeros((NBLK, BLK), jnp.float32)
    offs_acc = jnp.zeros((1, 128), jnp.float32)
    off = jnp.float32(0.0)
    for r in range(R):
        oh = (sv == r).astype(jnp.float32)
        excl = jnp.dot(oh, tri, preferred_element_type=jnp.float32)
        counts = jnp.sum(oh, axis=1, keepdims=True)
        base = jnp.dot(tri_blk, counts, preferred_element_type=jnp.float32)
        offs_acc = offs_acc + jnp.where(lane == r, off, 0.0)
        dest_acc = dest_acc + oh * (off + base + excl)
        off = off + jnp.sum(counts)
    offs_acc = offs_acc + jnp.where(lane == R, off, 0.0)
    dest_ref[...] = dest_acc.astype(jnp.int32)
    offs_ref[...] = offs_acc.astype(jnp.int32)

    bf = jnp.bfloat16
    h = jnp.dot(u_ref[...].astype(bf), w1u_ref[...],
                preferred_element_type=jnp.float32)
    h = h + jnp.dot(i_ref[...].astype(bf), w1i_ref[...],
                    preferred_element_type=jnp.float32)
    h = _leaky(h + b1_ref[...])
    ui = _leaky(jnp.dot(h.astype(bf), w2_ref[...],
                        preferred_element_type=jnp.float32) + b2_ref[...])
    ui_ref[...] = ui


def _group_body(offs_ref, xa_ref, xo_ref, w1a_ref, w1o_ref, b1_ref, w2_ref,
                b2_ref, out_ref):
    t = pl.program_id(0)
    row0 = t * MT
    rows = lax.broadcasted_iota(jnp.int32, (MT, 1), 0) + row0
    out_ref[...] = jnp.zeros((MT, H2), jnp.float32)
    for r in range(R):
        lo = offs_ref[r]
        hi = offs_ref[r + 1]

        @pl.when((lo < row0 + MT) & (hi > row0))
        def _():
            bf = jnp.bfloat16
            h = jnp.dot(xa_ref[...].astype(bf), w1a_ref[r],
                        preferred_element_type=jnp.float32)
            h = h + jnp.dot(xo_ref[...].astype(bf), w1o_ref[r],
                            preferred_element_type=jnp.float32)
            h = _leaky(h + b1_ref[r][None, :])
            g = _leaky(jnp.dot(h.astype(bf), w2_ref[r],
                               preferred_element_type=jnp.float32)
                       + b2_ref[r][None, :])
            mask = (rows >= lo) & (rows < hi)
            out_ref[...] = jnp.where(mask, g, out_ref[...])


def _dot_body(ui_ref, aos_ref, out_ref):
    aos = aos_ref[...].reshape(B, N, H2)
    ui = ui_ref[...]
    out_ref[...] = jnp.sum(aos * ui[:, None, :], axis=-1)


_sc_mesh = plsc.VectorSubcoreMesh(core_axis_name="c", subcore_axis_name="s")


@functools.partial(
    pl.kernel,
    out_type=(
        jax.ShapeDtypeStruct((T, D), jnp.float32),
        jax.ShapeDtypeStruct((T, D), jnp.float32),
    ),
    mesh=_sc_mesh,
    scratch_types=[
        pltpu.VMEM((_CH,), jnp.int32),
        pltpu.VMEM((_CH, D), jnp.float32),
        pltpu.SemaphoreType.DMA,
    ],
)
def _sc_scatter(a_hbm, o_hbm, dest_hbm, xa_hbm, xo_hbm, idx_v, rows_v, sem):
    wid = lax.axis_index("s") * _NC + lax.axis_index("c")
    base = wid * _PW
    for c in range(_PW // _CH):
        cb = base + c * _CH
        pltpu.sync_copy(dest_hbm.at[pl.ds(cb, _CH)], idx_v)
        pltpu.sync_copy(a_hbm.at[pl.ds(cb, _CH)], rows_v)
        pltpu.async_copy(rows_v, xa_hbm.at[idx_v], sem).wait()
        pltpu.sync_copy(o_hbm.at[pl.ds(cb, _CH)], rows_v)
        pltpu.async_copy(rows_v, xo_hbm.at[idx_v], sem).wait()


@functools.partial(
    pl.kernel,
    out_type=jax.ShapeDtypeStruct((T, H2), jnp.float32),
    mesh=_sc_mesh,
    scratch_types=[
        pltpu.VMEM((_CH,), jnp.int32),
        pltpu.VMEM((_CH, H2), jnp.float32),
        pltpu.SemaphoreType.DMA,
    ],
)
def _sc_gather(h2s_hbm, dest_hbm, aos_hbm, idx_v, rows_v, sem):
    wid = lax.axis_index("s") * _NC + lax.axis_index("c")
    base = wid * _PW
    for c in range(_PW // _CH):
        cb = base + c * _CH
        pltpu.sync_copy(dest_hbm.at[pl.ds(cb, _CH)], idx_v)
        pltpu.async_copy(h2s_hbm.at[idx_v], rows_v, sem).wait()
        pltpu.sync_copy(rows_v, aos_hbm.at[pl.ds(cb, _CH)])


def kernel(u_emb, i_emb, a_emb, o_emb, s,
           mlp_ao_W1, mlp_ao_b1, mlp_ao_W2, mlp_ao_b2,
           mlp_ui_W1, mlp_ui_b1, mlp_ui_W2, mlp_ui_b2):
    s2d = s.reshape(NBLK, BLK)
    bf = jnp.bfloat16
    w1u = mlp_ui_W1[:D].astype(bf)
    w1i = mlp_ui_W1[D:].astype(bf)
    w2ui = mlp_ui_W2.astype(bf)
    b1 = mlp_ui_b1.reshape(1, H1)
    b2 = mlp_ui_b2.reshape(1, H2)

    dest2d, offs, ui_emb = pl.pallas_call(
        _prep_body,
        out_shape=(
            jax.ShapeDtypeStruct((NBLK, BLK), jnp.int32),
            jax.ShapeDtypeStruct((1, 128), jnp.int32),
            jax.ShapeDtypeStruct((B, H2), jnp.float32),
        ),
    )(s2d, u_emb, i_emb, w1u, w1i, b1, w2ui, b2)

    dest = dest2d.reshape(T)
    a2 = a_emb.reshape(T, D)
    o2 = o_emb.reshape(T, D)

    xa, xo = _sc_scatter(a2, o2, dest)

    offs9 = offs[0, :R + 1]
    w1a = mlp_ao_W1[:, :D, :].astype(bf)
    w1o = mlp_ao_W1[:, D:, :].astype(bf)
    w2ao = mlp_ao_W2.astype(bf)

    h2s = pl.pallas_call(
        _group_body,
        grid=(NMT,),
        in_specs=[
            pl.BlockSpec(memory_space=pltpu.SMEM),
            pl.BlockSpec((MT, D), lambda t: (t, 0)),
            pl.BlockSpec((MT, D), lambda t: (t, 0)),
            pl.BlockSpec((R, D, H1), lambda t: (0, 0, 0)),
            pl.BlockSpec((R, D, H1), lambda t: (0, 0, 0)),
            pl.BlockSpec((R, H1), lambda t: (0, 0)),
            pl.BlockSpec((R, H1, H2), lambda t: (0, 0, 0)),
            pl.BlockSpec((R, H2), lambda t: (0, 0)),
        ],
        out_specs=pl.BlockSpec((MT, H2), lambda t: (t, 0)),
        out_shape=jax.ShapeDtypeStruct((T, H2), jnp.float32),
    )(offs9, xa, xo, w1a, w1o, mlp_ao_b1, w2ao, mlp_ao_b2)

    aos = _sc_gather(h2s, dest)

    pred = pl.pallas_call(
        _dot_body,
        out_shape=jax.ShapeDtypeStruct((B, N), jnp.float32),
    )(ui_emb, aos)
    return pred


# speedup vs baseline: 2.9584x; 1.0541x over previous
import functools

import jax
import jax.numpy as jnp
from jax import lax
from jax.experimental import pallas as pl
from jax.experimental.pallas import tpu as pltpu
from jax.experimental.pallas import tpu_sc as plsc

B, N, D, H1, H2, R = 1024, 8, 512, 512, 256, 8
T = B * N
BLK = 128
NBLK = T // BLK
MT = 256
NMT = T // MT

_NC, _NS = 2, 16
_NW = _NC * _NS
_PW = T // _NW
_CH = 128


def _leaky(x):
    return jnp.where(x >= 0, x, 0.01 * x)


def _route_body(s_ref, dest_ref, offs_ref):
    sv = s_ref[...]
    jj = lax.broadcasted_iota(jnp.int32, (BLK, BLK), 0)
    ii = lax.broadcasted_iota(jnp.int32, (BLK, BLK), 1)
    tri = (jj < ii).astype(jnp.float32)
    tj = lax.broadcasted_iota(jnp.int32, (NBLK, NBLK), 1)
    ti = lax.broadcasted_iota(jnp.int32, (NBLK, NBLK), 0)
    tri_blk = (tj < ti).astype(jnp.float32)
    lane = lax.broadcasted_iota(jnp.int32, (1, 128), 1)

    dest_acc = jnp.zeros((NBLK, BLK), jnp.float32)
    offs_acc = jnp.zeros((1, 128), jnp.float32)
    off = jnp.float32(0.0)
    for r in range(R):
        oh = (sv == r).astype(jnp.float32)
        excl = jnp.dot(oh, tri, preferred_element_type=jnp.float32)
        counts = jnp.sum(oh, axis=1, keepdims=True)
        base = jnp.dot(tri_blk, counts, preferred_element_type=jnp.float32)
        offs_acc = offs_acc + jnp.where(lane == r, off, 0.0)
        dest_acc = dest_acc + oh * (off + base + excl)
        off = off + jnp.sum(counts)
    offs_acc = offs_acc + jnp.where(lane == R, off, 0.0)
    dest_ref[...] = dest_acc.astype(jnp.int32)
    offs_ref[...] = offs_acc.astype(jnp.int32)


def _ui_body(u_ref, i_ref, w1u_ref, w1i_ref, b1_ref, w2_ref, b2_ref, ui_ref):
    bf = jnp.bfloat16
    h = jnp.dot(u_ref[...].astype(bf), w1u_ref[...],
                preferred_element_type=jnp.float32)
    h = h + jnp.dot(i_ref[...].astype(bf), w1i_ref[...],
                    preferred_element_type=jnp.float32)
    h = _leaky(h + b1_ref[...])
    ui = _leaky(jnp.dot(h.astype(bf), w2_ref[...],
                        preferred_element_type=jnp.float32) + b2_ref[...])
    ui_ref[...] = ui


def _group_body(offs_ref, xa_ref, xo_ref, w1a_ref, w1o_ref, b1_ref, w2_ref,
                b2_ref, out_ref):
    t = pl.program_id(0)
    row0 = t * MT
    rows = lax.broadcasted_iota(jnp.int32, (MT, 1), 0) + row0
    out_ref[...] = jnp.zeros((MT, H2), jnp.float32)
    for r in range(R):
        lo = offs_ref[r]
        hi = offs_ref[r + 1]

        @pl.when((lo < row0 + MT) & (hi > row0))
        def _():
            bf = jnp.bfloat16
            h = jnp.dot(xa_ref[...].astype(bf), w1a_ref[r],
                        preferred_element_type=jnp.float32)
            h = h + jnp.dot(xo_ref[...].astype(bf), w1o_ref[r],
                            preferred_element_type=jnp.float32)
            h = _leaky(h + b1_ref[r][None, :])
            g = _leaky(jnp.dot(h.astype(bf), w2_ref[r],
                               preferred_element_type=jnp.float32)
                       + b2_ref[r][None, :])
            mask = (rows >= lo) & (rows < hi)
            out_ref[...] = jnp.where(mask, g, out_ref[...])


def _dot_body(ui_ref, aos_ref, out_ref):
    aos = aos_ref[...].reshape(B, N, H2)
    ui = ui_ref[...]
    out_ref[...] = jnp.sum(aos * ui[:, None, :], axis=-1)


_sc_mesh = plsc.VectorSubcoreMesh(core_axis_name="c", subcore_axis_name="s")


_SCH = 64
_SNC = _PW // _SCH


@functools.partial(
    pl.kernel,
    out_type=(
        jax.ShapeDtypeStruct((T, D), jnp.float32),
        jax.ShapeDtypeStruct((T, D), jnp.float32),
    ),
    mesh=_sc_mesh,
    scratch_types=[
        pltpu.VMEM((_SNC, _SCH), jnp.int32),
        pltpu.VMEM((_SCH, D), jnp.float32),
        pltpu.VMEM((_SCH, D), jnp.float32),
        pltpu.SemaphoreType.DMA,
        pltpu.SemaphoreType.DMA,
    ],
)
def _sc_scatter(a_hbm, o_hbm, dest_hbm, xa_hbm, xo_hbm, idx_v, rows_a, rows_o,
                sem_a, sem_o):
    wid = lax.axis_index("s") * _NC + lax.axis_index("c")
    base = wid * _PW
    pend_a = pend_o = None
    for c in range(_SNC):
        cb = base + c * _SCH
        pltpu.sync_copy(dest_hbm.at[pl.ds(cb, _SCH)], idx_v.at[c])
        if pend_a is not None:
            pend_a.wait()
        pltpu.sync_copy(a_hbm.at[pl.ds(cb, _SCH)], rows_a)
        pend_a = pltpu.async_copy(rows_a, xa_hbm.at[idx_v.at[c]], sem_a)
        if pend_o is not None:
            pend_o.wait()
        pltpu.sync_copy(o_hbm.at[pl.ds(cb, _SCH)], rows_o)
        pend_o = pltpu.async_copy(rows_o, xo_hbm.at[idx_v.at[c]], sem_o)
    pend_a.wait()
    pend_o.wait()


@functools.partial(
    pl.kernel,
    out_type=jax.ShapeDtypeStruct((T, H2), jnp.float32),
    mesh=_sc_mesh,
    scratch_types=[
        pltpu.VMEM((2, _CH), jnp.int32),
        pltpu.VMEM((2, _CH, H2), jnp.float32),
        pltpu.SemaphoreType.DMA,
        pltpu.SemaphoreType.DMA,
    ],
)
def _sc_gather(h2s_hbm, dest_hbm, aos_hbm, idx_v, rows_v, sem0, sem1):
    wid = lax.axis_index("s") * _NC + lax.axis_index("c")
    base = wid * _PW
    sems = (sem0, sem1)
    pend = [None, None]
    for c in range(_PW // _CH):
        cb = base + c * _CH
        b = c % 2
        pltpu.sync_copy(dest_hbm.at[pl.ds(cb, _CH)], idx_v.at[b])
        if pend[b] is not None:
            pend[b][0].wait()
            pltpu.sync_copy(rows_v.at[pend[b][1] % 2], aos_hbm.at[pl.ds(pend[b][2], _CH)])
        pend[b] = (pltpu.async_copy(h2s_hbm.at[idx_v.at[b]], rows_v.at[b], sems[b]), c, cb)
    for b in (0, 1):
        if pend[b] is not None:
            pend[b][0].wait()
            pltpu.sync_copy(rows_v.at[b], aos_hbm.at[pl.ds(pend[b][2], _CH)])


def kernel(u_emb, i_emb, a_emb, o_emb, s,
           mlp_ao_W1, mlp_ao_b1, mlp_ao_W2, mlp_ao_b2,
           mlp_ui_W1, mlp_ui_b1, mlp_ui_W2, mlp_ui_b2):
    s2d = s.reshape(NBLK, BLK)
    bf = jnp.bfloat16
    w1u = mlp_ui_W1[:D].astype(bf)
    w1i = mlp_ui_W1[D:].astype(bf)
    w2ui = mlp_ui_W2.astype(bf)
    b1 = mlp_ui_b1.reshape(1, H1)
    b2 = mlp_ui_b2.reshape(1, H2)

    dest2d, offs = pl.pallas_call(
        _route_body,
        out_shape=(
            jax.ShapeDtypeStruct((NBLK, BLK), jnp.int32),
            jax.ShapeDtypeStruct((1, 128), jnp.int32),
        ),
    )(s2d)

    dest = dest2d.reshape(T)
    a2 = a_emb.reshape(T, D)
    o2 = o_emb.reshape(T, D)

    xa, xo = _sc_scatter(a2, o2, dest)

    ui_emb = pl.pallas_call(
        _ui_body,
        out_shape=jax.ShapeDtypeStruct((B, H2), jnp.float32),
    )(u_emb, i_emb, w1u, w1i, b1, w2ui, b2)

    offs9 = offs[0, :R + 1]
    w1a = mlp_ao_W1[:, :D, :].astype(bf)
    w1o = mlp_ao_W1[:, D:, :].astype(bf)
    w2ao = mlp_ao_W2.astype(bf)

    h2s = pl.pallas_call(
        _group_body,
        grid=(NMT,),
        in_specs=[
            pl.BlockSpec(memory_space=pltpu.SMEM),
            pl.BlockSpec((MT, D), lambda t: (t, 0)),
            pl.BlockSpec((MT, D), lambda t: (t, 0)),
            pl.BlockSpec((R, D, H1), lambda t: (0, 0, 0)),
            pl.BlockSpec((R, D, H1), lambda t: (0, 0, 0)),
            pl.BlockSpec((R, H1), lambda t: (0, 0)),
            pl.BlockSpec((R, H1, H2), lambda t: (0, 0, 0)),
            pl.BlockSpec((R, H2), lambda t: (0, 0)),
        ],
        out_specs=pl.BlockSpec((MT, H2), lambda t: (t, 0)),
        out_shape=jax.ShapeDtypeStruct((T, H2), jnp.float32),
    )(offs9, xa, xo, w1a, w1o, mlp_ao_b1, w2ao, mlp_ao_b2)

    aos = _sc_gather(h2s, dest)

    pred = pl.pallas_call(
        _dot_body,
        out_shape=jax.ShapeDtypeStruct((B, N), jnp.float32),
    )(ui_emb, aos)
    return pred
